# Initial kernel scaffold; baseline (speedup 1.0000x reference)
import jax
import jax.numpy as jnp
from jax.experimental import pallas as pl


def kernel(token_ids, embeddings):
    raise NotImplementedError("write your pallas kernel here")


# speedup vs baseline: 1.1021x; 1.1021x over previous
import functools

import jax
import jax.numpy as jnp
from jax import lax
from jax.experimental import pallas as pl
from jax.experimental.pallas import tpu as pltpu
from jax.experimental.pallas import tpu_sc as plsc

_NUM_EMB = 1000000
_D = 32
_B = 16384 * 50
_NW = 32
_B_PER_W = _B // _NW
_CHUNK = 1600
_NCHUNK = _B_PER_W // _CHUNK

_mesh = plsc.VectorSubcoreMesh(core_axis_name="c", subcore_axis_name="s")


@functools.partial(
    pl.kernel,
    mesh=_mesh,
    out_type=jax.ShapeDtypeStruct((_B, _D), jnp.float32),
    compiler_params=pltpu.CompilerParams(use_tc_tiling_on_sc=False),
    scratch_types=[
        pltpu.VMEM((_CHUNK,), jnp.int32),
        pltpu.VMEM((_CHUNK, _D), jnp.float32),
        pltpu.SemaphoreType.DMA,
    ],
)
def _gather_kernel(idx_hbm, table_hbm, out_hbm, idx_v, rows_v, sem):
    wid = lax.axis_index("s") * 2 + lax.axis_index("c")
    base = wid * _B_PER_W

    def body(i, carry):
        off = base + i * _CHUNK
        pltpu.sync_copy(idx_hbm.at[pl.ds(off, _CHUNK)], idx_v)
        pltpu.async_copy(table_hbm.at[idx_v], rows_v, sem).wait()
        pltpu.sync_copy(rows_v, out_hbm.at[pl.ds(off, _CHUNK)])
        return carry

    lax.fori_loop(0, _NCHUNK, body, 0)


def kernel(token_ids, embeddings):
    flat = token_ids.reshape(-1).astype(jnp.int32)
    out = _gather_kernel(flat, embeddings)
    return out.reshape(token_ids.shape + (_D,))


# speedup vs baseline: 1.1124x; 1.0093x over previous
import functools

import jax
import jax.numpy as jnp
from jax import lax
from jax.experimental import pallas as pl
from jax.experimental.pallas import tpu as pltpu
from jax.experimental.pallas import tpu_sc as plsc

_NUM_EMB = 1000000
_D = 32
_B = 16384 * 50
_NW = 32
_B_PER_W = _B // _NW
_CHUNK = 1600
_NCHUNK = _B_PER_W // _CHUNK

_mesh = plsc.VectorSubcoreMesh(core_axis_name="c", subcore_axis_name="s")


@functools.partial(
    pl.kernel,
    mesh=_mesh,
    out_type=jax.ShapeDtypeStruct((_B, _D), jnp.float32),
    compiler_params=pltpu.CompilerParams(use_tc_tiling_on_sc=False),
    scratch_types=[
        pltpu.VMEM((_NCHUNK, _CHUNK), jnp.int32),
        pltpu.VMEM((_CHUNK, _D), jnp.float32),
        pltpu.VMEM((_CHUNK, _D), jnp.float32),
        pltpu.SemaphoreType.DMA,
        pltpu.SemaphoreType.DMA,
        pltpu.SemaphoreType.DMA,
        pltpu.SemaphoreType.DMA,
    ],
)
def _gather_kernel(idx_hbm, table_hbm, out_hbm, idx_v, rows0, rows1,
                   sg0, sg1, so0, so1):
    wid = lax.axis_index("s") * 2 + lax.axis_index("c")
    base = wid * _B_PER_W
    rows = (rows0, rows1)
    sg = (sg0, sg1)
    so = (so0, so1)

    def gather_copy(g, b):
        return pltpu.make_async_copy(
            table_hbm.at[idx_v.at[g]], rows[b], sg[b])

    def out_copy(g, b):
        return pltpu.make_async_copy(
            rows[b], out_hbm.at[pl.ds(base + g * _CHUNK, _CHUNK)], so[b])

    pltpu.sync_copy(idx_hbm.at[wid], idx_v)
    gather_copy(0, 0).start()

    def body(p, carry):
        for b in range(2):
            g = 2 * p + b
            nb = 1 - b

            @pl.when(g >= 1)
            def _():
                out_copy(g - 1, nb).wait()

            @pl.when(g + 1 < _NCHUNK)
            def _():
                gather_copy(g + 1, nb).start()

            gather_copy(g, b).wait()
            out_copy(g, b).start()
        return carry

    lax.fori_loop(0, _NCHUNK // 2, body, 0)
    out_copy(_NCHUNK - 1, (_NCHUNK - 1) % 2).wait()


def kernel(token_ids, embeddings):
    flat = token_ids.reshape(_NW, _NCHUNK, _CHUNK).astype(jnp.int32)
    out = _gather_kernel(flat, embeddings)
    return out.reshape(token_ids.shape + (_D,))


# speedup vs baseline: 1.1128x; 1.0004x over previous
import functools

import jax
import jax.numpy as jnp
from jax import lax
from jax.experimental import pallas as pl
from jax.experimental.pallas import tpu as pltpu
from jax.experimental.pallas import tpu_sc as plsc

_NUM_EMB = 1000000
_D = 32
_B = 16384 * 50
_NW = 32
_B_PER_W = _B // _NW
_NBUF = 4
_CHUNK = 800
_NCHUNK = _B_PER_W // _CHUNK

_mesh = plsc.VectorSubcoreMesh(core_axis_name="c", subcore_axis_name="s")


@functools.partial(
    pl.kernel,
    mesh=_mesh,
    out_type=jax.ShapeDtypeStruct((_B, _D), jnp.float32),
    compiler_params=pltpu.CompilerParams(use_tc_tiling_on_sc=False),
    scratch_types=[
        pltpu.VMEM((_NCHUNK, _CHUNK), jnp.int32),
        *([pltpu.VMEM((_CHUNK, _D), jnp.float32)] * _NBUF),
        *([pltpu.SemaphoreType.DMA] * (2 * _NBUF)),
    ],
)
def _gather_kernel(idx_hbm, table_hbm, out_hbm, idx_v, *bufs_and_sems):
    rows = bufs_and_sems[:_NBUF]
    sg = bufs_and_sems[_NBUF:2 * _NBUF]
    so = bufs_and_sems[2 * _NBUF:]
    wid = lax.axis_index("s") * 2 + lax.axis_index("c")
    base = wid * _B_PER_W

    def gather_copy(g, b):
        return pltpu.make_async_copy(
            table_hbm.at[idx_v.at[g]], rows[b], sg[b])

    def out_copy(g, b):
        return pltpu.make_async_copy(
            rows[b], out_hbm.at[pl.ds(base + g * _CHUNK, _CHUNK)], so[b])

    pltpu.sync_copy(idx_hbm.at[wid], idx_v)
    for k in range(_NBUF - 2):
        gather_copy(k, k).start()

    def body(p, carry):
        for b in range(_NBUF):
            g = p * _NBUF + b
            f = g + _NBUF - 2
            fb = (b + _NBUF - 2) % _NBUF

            @pl.when(jnp.logical_and(f < _NCHUNK, g >= 2))
            def _():
                out_copy(g - 2, fb).wait()

            @pl.when(f < _NCHUNK)
            def _():
                gather_copy(f, fb).start()

            gather_copy(g, b).wait()
            out_copy(g, b).start()
        return carry

    lax.fori_loop(0, _NCHUNK // _NBUF, body, 0)
    for k in range(_NCHUNK - _NBUF, _NCHUNK):
        out_copy(k, k % _NBUF).wait()


def kernel(token_ids, embeddings):
    flat = token_ids.reshape(_NW, _NCHUNK, _CHUNK).astype(jnp.int32)
    out = _gather_kernel(flat, embeddings)
    return out.reshape(token_ids.shape + (_D,))


# speedup vs baseline: 1.8044x; 1.6215x over previous
import functools

import jax
import jax.numpy as jnp
from jax import lax
from jax.experimental import pallas as pl
from jax.experimental.pallas import tpu as pltpu
from jax.experimental.pallas import tpu_sc as plsc

_NUM_EMB = 1000000
_D = 32
_S = 50
_BATCH = 16384
_NW = 32
_R_PER_W = _BATCH // _NW
_NBUF = 4
_CH = 16
_NCHUNK = _R_PER_W // _CH

_mesh = plsc.VectorSubcoreMesh(core_axis_name="c", subcore_axis_name="s")


@functools.partial(
    pl.kernel,
    mesh=_mesh,
    out_type=jax.ShapeDtypeStruct((_BATCH, _S, _D), jnp.float32),
    compiler_params=pltpu.CompilerParams(use_tc_tiling_on_sc=False),
    scratch_types=[
        pltpu.VMEM((_R_PER_W, _S), jnp.int32),
        *([pltpu.VMEM((_CH, _S, _D), jnp.float32)] * _NBUF),
        *([pltpu.SemaphoreType.DMA] * (2 * _NBUF)),
    ],
)
def _gather_kernel(idx_hbm, table_hbm, out_hbm, idx_v, *bufs_and_sems):
    rows = bufs_and_sems[:_NBUF]
    sg = bufs_and_sems[_NBUF:2 * _NBUF]
    so = bufs_and_sems[2 * _NBUF:]
    wid = lax.axis_index("s") * 2 + lax.axis_index("c")
    base = wid * _R_PER_W

    def gather_chunk(g, b):
        for r in range(_CH):
            pltpu.async_copy(
                table_hbm.at[idx_v.at[g * _CH + r]], rows[b].at[r], sg[b])

    def gather_wait(g, b):
        pltpu.make_async_copy(
            out_hbm.at[pl.ds(base + g * _CH, _CH)], rows[b], sg[b]).wait()

    def out_copy(g, b):
        return pltpu.make_async_copy(
            rows[b], out_hbm.at[pl.ds(base + g * _CH, _CH)], so[b])

    pltpu.sync_copy(idx_hbm.at[wid], idx_v)
    for k in range(_NBUF - 2):
        gather_chunk(k, k)

    def body(p, carry):
        for b in range(_NBUF):
            g = p * _NBUF + b
            f = g + _NBUF - 2
            fb = (b + _NBUF - 2) % _NBUF

            @pl.when(jnp.logical_and(f < _NCHUNK, g >= 2))
            def _():
                out_copy(g - 2, fb).wait()

            @pl.when(f < _NCHUNK)
            def _():
                gather_chunk(f, fb)

            gather_wait(g, b)
            out_copy(g, b).start()
        return carry

    lax.fori_loop(0, _NCHUNK // _NBUF, body, 0)
    for k in range(_NCHUNK - _NBUF, _NCHUNK):
        out_copy(k, k % _NBUF).wait()


def kernel(token_ids, embeddings):
    idx = token_ids.reshape(_NW, _R_PER_W, _S).astype(jnp.int32)
    return _gather_kernel(idx, embeddings)


# speedup vs baseline: 1.8055x; 1.0006x over previous
import functools

import jax
import jax.numpy as jnp
from jax import lax
from jax.experimental import pallas as pl
from jax.experimental.pallas import tpu as pltpu
from jax.experimental.pallas import tpu_sc as plsc

_NUM_EMB = 1000000
_D = 32
_S = 50
_BATCH = 16384
_NW = 32
_R_PER_W = _BATCH // _NW
_NBUF = 4
_CH = 16
_NCHUNK = _R_PER_W // _CH

_mesh = plsc.VectorSubcoreMesh(core_axis_name="c", subcore_axis_name="s")


@functools.partial(
    pl.kernel,
    mesh=_mesh,
    out_type=jax.ShapeDtypeStruct((_BATCH, _S, _D), jnp.float32),
    compiler_params=pltpu.CompilerParams(use_tc_tiling_on_sc=False),
    scratch_types=[
        pltpu.VMEM((_R_PER_W, _S), jnp.int32),
        *([pltpu.VMEM((_CH, _S, _D), jnp.float32)] * _NBUF),
        *([pltpu.SemaphoreType.DMA] * (2 * _NBUF)),
    ],
)
def _gather_kernel(idx_hbm, table_hbm, out_hbm, idx_v, *bufs_and_sems):
    rows = bufs_and_sems[:_NBUF]
    sg = bufs_and_sems[_NBUF:2 * _NBUF]
    so = bufs_and_sems[2 * _NBUF:]
    wid = lax.axis_index("s") * 2 + lax.axis_index("c")
    base = wid * _R_PER_W

    def gather_chunk(g, b):
        for r in range(_CH):
            pltpu.async_copy(
                table_hbm.at[idx_v.at[g * _CH + r]], rows[b].at[r], sg[b])

    def gather_wait(g, b):
        pltpu.make_async_copy(
            out_hbm.at[pl.ds(base + g * _CH, _CH)], rows[b], sg[b]).wait()

    def out_copy(g, b):
        return pltpu.make_async_copy(
            rows[b], out_hbm.at[pl.ds(base + g * _CH, _CH)], so[b])

    pltpu.sync_copy(idx_hbm.at[pl.ds(base, _R_PER_W)], idx_v)
    for k in range(_NBUF - 2):
        gather_chunk(k, k)

    def body(p, carry):
        for b in range(_NBUF):
            g = p * _NBUF + b
            f = g + _NBUF - 2
            fb = (b + _NBUF - 2) % _NBUF

            @pl.when(jnp.logical_and(f < _NCHUNK, g >= 2))
            def _():
                out_copy(g - 2, fb).wait()

            @pl.when(f < _NCHUNK)
            def _():
                gather_chunk(f, fb)

            gather_wait(g, b)
            out_copy(g, b).start()
        return carry

    lax.fori_loop(0, _NCHUNK // _NBUF, body, 0)
    for k in range(_NCHUNK - _NBUF, _NCHUNK):
        out_copy(k, k % _NBUF).wait()


def kernel(token_ids, embeddings):
    return _gather_kernel(token_ids.astype(jnp.int32), embeddings)


# speedup vs baseline: 1.8129x; 1.0041x over previous
import functools

import jax
import jax.numpy as jnp
from jax import lax
from jax.experimental import pallas as pl
from jax.experimental.pallas import tpu as pltpu
from jax.experimental.pallas import tpu_sc as plsc

_NUM_EMB = 1000000
_D = 32
_S = 50
_BATCH = 16384
_NW = 32
_B_PER_W = _BATCH // _NW
_NBUF = 5

_mesh = plsc.VectorSubcoreMesh(core_axis_name="c", subcore_axis_name="s")


@functools.partial(
    pl.kernel,
    mesh=_mesh,
    out_type=jax.ShapeDtypeStruct((_BATCH, _S, _D), jnp.float32),
    compiler_params=pltpu.CompilerParams(use_tc_tiling_on_sc=False),
    scratch_types=[
        pltpu.VMEM((_S, _B_PER_W), jnp.int32),
        *([pltpu.VMEM((_B_PER_W, _D), jnp.float32)] * _NBUF),
        *([pltpu.SemaphoreType.DMA] * (2 * _NBUF)),
    ],
)
def _gather_kernel(idxt_hbm, table_hbm, out_hbm, idx_v, *bufs_and_sems):
    rows = bufs_and_sems[:_NBUF]
    sg = bufs_and_sems[_NBUF:2 * _NBUF]
    so = bufs_and_sems[2 * _NBUF:]
    wid = lax.axis_index("s") * 2 + lax.axis_index("c")
    base = wid * _B_PER_W

    def gather_copy(p, b):
        return pltpu.make_async_copy(
            table_hbm.at[idx_v.at[p]], rows[b], sg[b])

    def out_copy(p, b):
        return pltpu.make_async_copy(
            rows[b], out_hbm.at[pl.ds(base, _B_PER_W), p], so[b])

    pltpu.sync_copy(idxt_hbm.at[:, pl.ds(base, _B_PER_W)], idx_v)
    for k in range(_NBUF - 2):
        gather_copy(k, k).start()

    def body(q, carry):
        for b in range(_NBUF):
            g = q * _NBUF + b
            f = g + _NBUF - 2
            fb = (b + _NBUF - 2) % _NBUF

            @pl.when(jnp.logical_and(f < _S, g >= 2))
            def _():
                out_copy(g - 2, fb).wait()

            @pl.when(f < _S)
            def _():
                gather_copy(f, fb).start()

            gather_copy(g, b).wait()
            out_copy(g, b).start()
        return carry

    lax.fori_loop(0, _S // _NBUF, body, 0)
    for k in range(_S - _NBUF, _S):
        out_copy(k, k % _NBUF).wait()


def kernel(token_ids, embeddings):
    return _gather_kernel(token_ids.T.astype(jnp.int32), embeddings)
